# baseline (speedup 1.0000x reference)
import jax
import jax.numpy as jnp
from jax import lax
from jax.experimental import pallas as pl
from jax.experimental.pallas import tpu as pltpu
from jax.experimental.pallas import tpu_sc as plsc

N_PREV = 50000
N_OUT = 10000
S = 32
D = 128
W = D // 2
L = 16
NC, NS = 2, 16
NW = NC * NS
C = 8
HALF = 128
NH = C * S // HALF
N_CHUNKS = N_OUT // C
NKP = (N_CHUNKS + NW - 1) // NW
NKP += NKP % 2
GROUPS = D // L
UNROLL = 4


def _body(feat_hbm, idx_hbm, out_hbm, idx_v, rows_v, out_v, gs0, gs1, os0, os1):
    wid = lax.axis_index("s") * NC + lax.axis_index("c")
    gsem = (gs0, gs1)
    osem = (os0, os1)
    lane = lax.iota(jnp.int32, L)
    pair = lane >> 1
    even = (lane & 1) == 0

    def chunk_of(m):
        return wid + m * NW

    def start_gather(m, b):
        pltpu.sync_copy(idx_hbm.at[pl.ds(chunk_of(m) * NH, NH)],
                        idx_v.at[b])
        for h in range(NH):
            pltpu.async_copy(feat_hbm.at[idx_v.at[b, h]], rows_v.at[b, h],
                             gsem[b])

    def wait_gather(b):
        for h in range(NH):
            pltpu.make_async_copy(feat_hbm.at[idx_v.at[b, h]],
                                  rows_v.at[b, h], gsem[b]).wait()

    def wait_out(m, b):
        pltpu.make_async_copy(out_v.at[b],
                              out_hbm.at[pl.ds(chunk_of(m) * C, C)],
                              osem[b]).wait()

    dnums = lax.GatherDimensionNumbers(
        offset_dims=(), collapsed_slice_dims=(0,), start_index_map=(0,))

    def lane_gather(x, idx):
        return lax.gather(x, idx[:, None], dimension_numbers=dnums,
                          slice_sizes=(1,),
                          mode=lax.GatherScatterMode.PROMISE_IN_BOUNDS)

    def interleave(e, o, half):
        idx = pair + 8 * half
        return jnp.where(even, lane_gather(e, idx), lane_gather(o, idx))

    def compute_store(m, b):
        for n in range(C):
            h = n // (HALF // S)
            r0 = (n % (HALF // S)) * S

            def s_body(t, accs, h=h, r0=r0):
                for dt in range(UNROLL):
                    r = r0 + t * UNROLL + dt
                    new = []
                    for g in range(W // L):
                        w = rows_v[b, h, r, pl.ds(g * L, L)]
                        lo = lax.bitcast_convert_type(w << 16, jnp.float32)
                        hi = lax.bitcast_convert_type(
                            w & jnp.int32(-65536), jnp.float32)
                        new.append(accs[2 * g] + lo)
                        new.append(accs[2 * g + 1] + hi)
                    accs = tuple(new)
                return accs

            accs = tuple(jnp.zeros((L,), jnp.float32)
                         for _ in range(2 * (W // L)))
            accs = lax.fori_loop(0, S // UNROLL, s_body, accs)
            for g in range(W // L):
                e = accs[2 * g] * (1.0 / S)
                o = accs[2 * g + 1] * (1.0 / S)
                out_v[b, n, pl.ds(g * 2 * L, L)] = interleave(e, o, 0)
                out_v[b, n, pl.ds(g * 2 * L + L, L)] = interleave(e, o, 1)
        pltpu.async_copy(out_v.at[b], out_hbm.at[pl.ds(chunk_of(m) * C, C)],
                         osem[b])

    start_gather(0, 0)

    def j_body(i, carry):
        j = i * 2
        for b in range(2):
            m = j + b

            @pl.when(chunk_of(m + 1) < N_CHUNKS)
            def _(m=m, b=b):
                start_gather(m + 1, 1 - b)

            @pl.when(chunk_of(m) < N_CHUNKS)
            def _(m=m, b=b):
                wait_gather(b)

                @pl.when(m >= 2)
                def _(m=m, b=b):
                    wait_out(m - 2, b)

                compute_store(m, b)

        return carry

    lax.fori_loop(0, NKP // 2, j_body, 0)

    for b in range(2):
        wait_out(b, b)


@jax.jit
def kernel(features, neighbor_idx):
    feat_words = lax.bitcast_convert_type(
        features.astype(jnp.bfloat16).reshape(N_PREV, W, 2), jnp.int32)
    idx2 = neighbor_idx.reshape(N_CHUNKS * NH, HALF).astype(jnp.int32)
    mesh = plsc.VectorSubcoreMesh(core_axis_name="c", subcore_axis_name="s")
    run = pl.kernel(
        _body,
        out_type=jax.ShapeDtypeStruct((N_OUT, D), jnp.float32),
        mesh=mesh,
        compiler_params=pltpu.CompilerParams(use_tc_tiling_on_sc=False),
        scratch_types=[
            pltpu.VMEM((2, NH, HALF), jnp.int32),
            pltpu.VMEM((2, NH, HALF, W), jnp.int32),
            pltpu.VMEM((2, C, D), jnp.float32),
            pltpu.SemaphoreType.DMA,
            pltpu.SemaphoreType.DMA,
            pltpu.SemaphoreType.DMA,
            pltpu.SemaphoreType.DMA,
        ],
    )
    return run(feat_words, idx2)


# speedup vs baseline: 3.3245x; 3.3245x over previous
import jax
import jax.numpy as jnp
from jax import lax
from jax.experimental import pallas as pl
from jax.experimental.pallas import tpu as pltpu
from jax.experimental.pallas import tpu_sc as plsc

N_PREV = 50000
N_OUT = 10000
S = 32
D = 128
L = 16
NC, NS = 2, 16
NW = NC * NS
C = 8
HALF = 128
NH = C * S // HALF
N_CHUNKS = N_OUT // C
NKP = (N_CHUNKS + NW - 1) // NW
NKP += NKP % 2
GROUPS = D // L
UNROLL = 4


def _body(feat_hbm, idx_hbm, out_hbm, idx_v, rows_v, out_v, gs0, gs1, os0, os1):
    wid = lax.axis_index("s") * NC + lax.axis_index("c")
    gsem = (gs0, gs1)
    osem = (os0, os1)

    def chunk_of(m):
        return wid + m * NW

    def start_gather(m, b):
        pltpu.sync_copy(idx_hbm.at[chunk_of(m)], idx_v.at[b])
        for h in range(NH):
            pltpu.async_copy(feat_hbm.at[idx_v.at[b, h]], rows_v.at[b, h],
                             gsem[b])

    def wait_gather(b):
        for h in range(NH):
            pltpu.make_async_copy(feat_hbm.at[idx_v.at[b, h]],
                                  rows_v.at[b, h], gsem[b]).wait()

    def wait_out(m, b):
        pltpu.make_async_copy(out_v.at[b],
                              out_hbm.at[pl.ds(chunk_of(m) * C, C)],
                              osem[b]).wait()

    def compute_store(m, b):
        for n in range(C):
            h = n // (HALF // S)
            r0 = (n % (HALF // S)) * S

            def s_body(t, accs, h=h, r0=r0):
                for dt in range(UNROLL):
                    r = r0 + t * UNROLL + dt
                    accs = tuple(accs[g] + rows_v[b, h, r, pl.ds(g * L, L)]
                                 for g in range(GROUPS))
                return accs

            accs = tuple(jnp.zeros((L,), jnp.float32) for _ in range(GROUPS))
            accs = lax.fori_loop(0, S // UNROLL, s_body, accs)
            for g in range(GROUPS):
                out_v[b, n, pl.ds(g * L, L)] = accs[g] * (1.0 / S)
        pltpu.async_copy(out_v.at[b], out_hbm.at[pl.ds(chunk_of(m) * C, C)],
                         osem[b])

    start_gather(0, 0)

    def j_body(i, carry):
        j = i * 2
        for b in range(2):
            m = j + b

            @pl.when(chunk_of(m + 1) < N_CHUNKS)
            def _(m=m, b=b):
                start_gather(m + 1, 1 - b)

            @pl.when(chunk_of(m) < N_CHUNKS)
            def _(m=m, b=b):
                wait_gather(b)

                @pl.when(m >= 2)
                def _(m=m, b=b):
                    wait_out(m - 2, b)

                compute_store(m, b)

        return carry

    lax.fori_loop(0, NKP // 2, j_body, 0)

    for b in range(2):
        wait_out(b, b)


@jax.jit
def kernel(features, neighbor_idx):
    idx3 = neighbor_idx.reshape(N_CHUNKS, NH, HALF).astype(jnp.int32)
    mesh = plsc.VectorSubcoreMesh(core_axis_name="c", subcore_axis_name="s")
    run = pl.kernel(
        _body,
        out_type=jax.ShapeDtypeStruct((N_OUT, D), jnp.float32),
        mesh=mesh,
        scratch_types=[
            pltpu.VMEM((2, NH, HALF), jnp.int32),
            pltpu.VMEM((2, NH, HALF, D), jnp.float32),
            pltpu.VMEM((2, C, D), jnp.float32),
            pltpu.SemaphoreType.DMA,
            pltpu.SemaphoreType.DMA,
            pltpu.SemaphoreType.DMA,
            pltpu.SemaphoreType.DMA,
        ],
    )
    return run(features, idx3)


# speedup vs baseline: 3.3425x; 1.0054x over previous
import jax
import jax.numpy as jnp
from jax import lax
from jax.experimental import pallas as pl
from jax.experimental.pallas import tpu as pltpu
from jax.experimental.pallas import tpu_sc as plsc

N_PREV = 50000
N_OUT = 10000
S = 32
D = 128
L = 16
NC, NS = 2, 16
NW = NC * NS
C = 8
HALF = 128
NH = C * S // HALF
N_CHUNKS = N_OUT // C
NKP = (N_CHUNKS + NW - 1) // NW
NKP += NKP % 2
GROUPS = D // L
UNROLL = 4


def _body(feat_hbm, idx_hbm, out_hbm, idx_v, rows_v, out_v, gs0, gs1, os0, os1):
    wid = lax.axis_index("s") * NC + lax.axis_index("c")
    gsem = (gs0, gs1)
    osem = (os0, os1)

    def chunk_of(m):
        return wid + m * NW

    def start_gather(m, b):
        pltpu.sync_copy(idx_hbm.at[pl.ds(chunk_of(m) * NH, NH)],
                        idx_v.at[b])
        for h in range(NH):
            pltpu.async_copy(feat_hbm.at[idx_v.at[b, h]], rows_v.at[b, h],
                             gsem[b])

    def wait_gather(b):
        for h in range(NH):
            pltpu.make_async_copy(feat_hbm.at[idx_v.at[b, h]],
                                  rows_v.at[b, h], gsem[b]).wait()

    def wait_out(m, b):
        pltpu.make_async_copy(out_v.at[b],
                              out_hbm.at[pl.ds(chunk_of(m) * C, C)],
                              osem[b]).wait()

    def compute_store(m, b):
        for n in range(C):
            h = n // (HALF // S)
            r0 = (n % (HALF // S)) * S

            def s_body(t, accs, h=h, r0=r0):
                for dt in range(UNROLL):
                    r = r0 + t * UNROLL + dt
                    accs = tuple(accs[g] + rows_v[b, h, r, pl.ds(g * L, L)]
                                 for g in range(GROUPS))
                return accs

            accs = tuple(jnp.zeros((L,), jnp.float32) for _ in range(GROUPS))
            accs = lax.fori_loop(0, S // UNROLL, s_body, accs)
            for g in range(GROUPS):
                out_v[b, n, pl.ds(g * L, L)] = accs[g] * (1.0 / S)
        pltpu.async_copy(out_v.at[b], out_hbm.at[pl.ds(chunk_of(m) * C, C)],
                         osem[b])

    start_gather(0, 0)

    def j_body(i, carry):
        j = i * 2
        for b in range(2):
            m = j + b

            @pl.when(chunk_of(m + 1) < N_CHUNKS)
            def _(m=m, b=b):
                start_gather(m + 1, 1 - b)

            @pl.when(chunk_of(m) < N_CHUNKS)
            def _(m=m, b=b):
                wait_gather(b)

                @pl.when(m >= 2)
                def _(m=m, b=b):
                    wait_out(m - 2, b)

                compute_store(m, b)

        return carry

    lax.fori_loop(0, NKP // 2, j_body, 0)

    for b in range(2):
        wait_out(b, b)


@jax.jit
def kernel(features, neighbor_idx):
    idx2 = neighbor_idx.reshape(N_CHUNKS * NH, HALF).astype(jnp.int32)
    mesh = plsc.VectorSubcoreMesh(core_axis_name="c", subcore_axis_name="s")
    run = pl.kernel(
        _body,
        out_type=jax.ShapeDtypeStruct((N_OUT, D), jnp.float32),
        mesh=mesh,
        scratch_types=[
            pltpu.VMEM((2, NH, HALF), jnp.int32),
            pltpu.VMEM((2, NH, HALF, D), jnp.float32),
            pltpu.VMEM((2, C, D), jnp.float32),
            pltpu.SemaphoreType.DMA,
            pltpu.SemaphoreType.DMA,
            pltpu.SemaphoreType.DMA,
            pltpu.SemaphoreType.DMA,
        ],
    )
    return run(features, idx2)
